# Initial kernel scaffold; baseline (speedup 1.0000x reference)
import jax
import jax.numpy as jnp
from jax.experimental import pallas as pl


def kernel(x, table, W, b):
    raise NotImplementedError("write your pallas kernel here")


# speedup vs baseline: 5.6363x; 5.6363x over previous
import functools

import jax
import jax.numpy as jnp
from jax import lax
from jax.experimental import pallas as pl
from jax.experimental.pallas import tpu as pltpu
from jax.experimental.pallas import tpu_sc as plsc

_B = 16384
_L = 200
_EMB = 16
_NC = 2
_NS = 16
_NW = _NC * _NS
_SPW = _B // _NW
_G = 16
_NCH = _SPW // _G
_IDX_CH = _G * _L
_GR = 128
_NG = _IDX_CH // _GR


def _sc_body(x_ref, tab_ref, w_ref, b_ref, out_ref,
             idx_v, rows_v, wv_v, bv_v, out_v, sem):
    wid = lax.axis_index("s") * _NC + lax.axis_index("c")
    pltpu.sync_copy(w_ref, wv_v)
    pltpu.sync_copy(b_ref, bv_v)
    wv = wv_v[...]

    def chunk_body(c, carry):
        base = wid * (_SPW * _L) + c * _IDX_CH
        pltpu.sync_copy(x_ref.at[pl.ds(base, _IDX_CH)], idx_v)
        descs = [
            pltpu.async_copy(tab_ref.at[idx_v.at[pl.ds(j * _GR, _GR)]],
                             rows_v.at[pl.ds(j * _GR, _GR), :], sem)
            for j in range(_NG)
        ]
        for d in descs:
            d.wait()

        lanes = lax.iota(jnp.int32, 16)

        def samp_body(s, zvec):
            rb = s * _L

            def red(j, acc):
                return acc + rows_v[rb + j, :]

            acc = lax.fori_loop(0, _L, red, jnp.zeros((_EMB,), jnp.float32))
            z = jnp.sum(acc * wv)
            return jnp.where(lanes == s, jnp.full((16,), z), zvec)

        zvec = lax.fori_loop(0, _G, samp_body, jnp.zeros((16,), jnp.float32))
        out_v[pl.ds(c * _G, _G)] = zvec
        return carry

    lax.fori_loop(0, _NCH, chunk_body, 0)

    bv = bv_v[...]
    for g in range(_SPW // 16):
        z = out_v[pl.ds(g * 16, 16)] + bv
        out_v[pl.ds(g * 16, 16)] = 1.0 / (1.0 + jnp.exp(-z))
    pltpu.sync_copy(out_v, out_ref.at[pl.ds(wid * _SPW, _SPW)])


_sc_call = functools.partial(
    pl.kernel,
    out_type=jax.ShapeDtypeStruct((_B,), jnp.float32),
    mesh=plsc.VectorSubcoreMesh(core_axis_name="c", subcore_axis_name="s"),
    compiler_params=pltpu.CompilerParams(
        needs_layout_passes=False, use_tc_tiling_on_sc=False),
    scratch_types=[
        pltpu.VMEM((_IDX_CH,), jnp.int32),
        pltpu.VMEM((_IDX_CH, _EMB), jnp.float32),
        pltpu.VMEM((_EMB,), jnp.float32),
        pltpu.VMEM((_EMB,), jnp.float32),
        pltpu.VMEM((_SPW,), jnp.float32),
        pltpu.SemaphoreType.DMA,
    ],
)(_sc_body)


def kernel(x, table, W, b):
    xf = x.reshape(-1).astype(jnp.int32)
    wv = W[0].astype(jnp.float32) / jnp.float32(_L)
    bv = jnp.broadcast_to(b.astype(jnp.float32), (_EMB,))
    out = _sc_call(xf, table, wv, bv)
    return out.reshape(_B, 1)


# speedup vs baseline: 9.2115x; 1.6343x over previous
import functools

import jax
import jax.numpy as jnp
from jax import lax
from jax.experimental import pallas as pl
from jax.experimental.pallas import tpu as pltpu
from jax.experimental.pallas import tpu_sc as plsc

_B = 16384
_L = 200
_EMB = 16
_NC = 2
_NS = 16
_NW = _NC * _NS
_SPW = _B // _NW
_G = 16
_NCH = _SPW // _G
_IDX_CH = _G * _L
_GR = 128
_NG = _IDX_CH // _GR


def _sc_body(x_ref, tab_ref, w_ref, b_ref, out_ref,
             idx0, idx1, rows0, rows1, wv_v, bv_v, out_v, sem0, sem1):
    wid = lax.axis_index("s") * _NC + lax.axis_index("c")
    xbase = wid * (_SPW * _L)
    pltpu.sync_copy(w_ref, wv_v)
    pltpu.sync_copy(b_ref, bv_v)
    wv = wv_v[...]
    lanes = lax.iota(jnp.int32, 16)
    bufs = ((idx0, rows0, sem0), (idx1, rows1, sem1))

    def fetch(c, buf):
        idx_v, rows_v, sem = bufs[buf]
        pltpu.sync_copy(x_ref.at[pl.ds(xbase + c * _IDX_CH, _IDX_CH)], idx_v)
        for j in range(_NG):
            pltpu.async_copy(tab_ref.at[idx_v.at[pl.ds(j * _GR, _GR)]],
                             rows_v.at[pl.ds(j * _GR, _GR), :], sem)

    def drain(buf):
        _, rows_v, sem = bufs[buf]
        pltpu.make_async_copy(tab_ref.at[pl.ds(0, _IDX_CH), :], rows_v,
                              sem).wait()

    def compute(c, buf):
        _, rows_v, _ = bufs[buf]

        def samp_body(s, zvec):
            rb = s * _L

            def red(j, acc):
                return acc + rows_v[rb + j, :]

            acc = lax.fori_loop(0, _L, red, jnp.zeros((_EMB,), jnp.float32),
                                unroll=10)
            z = jnp.sum(acc * wv)
            return jnp.where(lanes == s, jnp.full((16,), z), zvec)

        zvec = lax.fori_loop(0, _G, samp_body, jnp.zeros((16,), jnp.float32))
        out_v[pl.ds(c * _G, _G)] = zvec

    fetch(0, 0)

    def pair_body(cp, carry):
        c0 = cp * 2
        drain(0)
        fetch(c0 + 1, 1)
        compute(c0, 0)
        drain(1)

        @pl.when(cp < _NCH // 2 - 1)
        def _():
            fetch(c0 + 2, 0)

        compute(c0 + 1, 1)
        return carry

    lax.fori_loop(0, _NCH // 2, pair_body, 0)

    bv = bv_v[...]
    for g in range(_SPW // 16):
        z = out_v[pl.ds(g * 16, 16)] + bv
        out_v[pl.ds(g * 16, 16)] = 1.0 / (1.0 + jnp.exp(-z))
    pltpu.sync_copy(out_v, out_ref.at[pl.ds(wid * _SPW, _SPW)])


_sc_call = functools.partial(
    pl.kernel,
    out_type=jax.ShapeDtypeStruct((_B,), jnp.float32),
    mesh=plsc.VectorSubcoreMesh(core_axis_name="c", subcore_axis_name="s"),
    compiler_params=pltpu.CompilerParams(
        needs_layout_passes=False, use_tc_tiling_on_sc=False),
    scratch_types=[
        pltpu.VMEM((_IDX_CH,), jnp.int32),
        pltpu.VMEM((_IDX_CH,), jnp.int32),
        pltpu.VMEM((_IDX_CH, _EMB), jnp.float32),
        pltpu.VMEM((_IDX_CH, _EMB), jnp.float32),
        pltpu.VMEM((_EMB,), jnp.float32),
        pltpu.VMEM((_EMB,), jnp.float32),
        pltpu.VMEM((_SPW,), jnp.float32),
        pltpu.SemaphoreType.DMA,
        pltpu.SemaphoreType.DMA,
    ],
)(_sc_body)


def kernel(x, table, W, b):
    xf = x.reshape(-1).astype(jnp.int32)
    wv = W[0].astype(jnp.float32) / jnp.float32(_L)
    bv = jnp.broadcast_to(b.astype(jnp.float32), (_EMB,))
    out = _sc_call(xf, table, wv, bv)
    return out.reshape(_B, 1)


# speedup vs baseline: 32.6940x; 3.5493x over previous
import functools

import jax
import jax.numpy as jnp
from jax import lax
from jax.experimental import pallas as pl
from jax.experimental.pallas import tpu as pltpu
from jax.experimental.pallas import tpu_sc as plsc

_B = 16384
_L = 200
_EMB = 16
_V = 1_000_000
_NC = 2
_NS = 16
_NW = _NC * _NS
_SPW = _B // _NW
_GW = 128
_NCH = _SPW // _GW
_NGRP = _GW // 16
_TWC = 131072


def _tw_body(tabT_ref, w_ref, tw_ref):
    w = w_ref[:, 0:1]
    tw_ref[...] = jnp.sum(tabT_ref[...] * w, axis=0)


_tw_call = pl.pallas_call(
    _tw_body,
    grid=(pl.cdiv(_V, _TWC),),
    in_specs=[
        pl.BlockSpec((_EMB, _TWC), lambda j: (0, j)),
        pl.BlockSpec((_EMB, 128), lambda j: (0, 0)),
    ],
    out_specs=pl.BlockSpec((_TWC,), lambda j: (j,)),
    out_shape=jax.ShapeDtypeStruct((_V,), jnp.float32),
)


def _sc_body(xt_ref, tw_ref, b_ref, out_ref,
             idx0, idx1, rows0, rows1, bv_v, out_v,
             semi0, semi1, semg0, semg1):
    wid = lax.axis_index("s") * _NC + lax.axis_index("c")
    sbase = wid * _SPW
    pltpu.sync_copy(b_ref, bv_v)
    idxs = (idx0, idx1)
    rows = (rows0, rows1)
    semis = (semi0, semi1)
    semgs = (semg0, semg1)

    def issue_idx(c, buf):
        pltpu.async_copy(
            xt_ref.at[:, pl.ds(sbase + c * _GW, _GW)], idxs[buf], semis[buf])

    def wait_idx(c, buf):
        pltpu.make_async_copy(
            xt_ref.at[:, pl.ds(sbase + c * _GW, _GW)], idxs[buf],
            semis[buf]).wait()

    def issue_gathers(buf):
        def issue(j, carry):
            pltpu.async_copy(
                tw_ref.at[idxs[buf].at[j, :]],
                rows[buf].at[j, :], semgs[buf])
            return carry

        lax.fori_loop(0, _L, issue, 0, unroll=8)

    def drain_gathers(buf):
        pltpu.make_async_copy(
            xt_ref.at[:, pl.ds(sbase, _GW)], rows[buf], semgs[buf]).wait()

    def compute(c, buf):
        rv = rows[buf]

        def red(k, accs):
            return tuple(accs[q] + rv[k, pl.ds(q * 16, 16)]
                         for q in range(_NGRP))

        zero = jnp.zeros((16,), jnp.float32)
        accs = lax.fori_loop(0, _L, red, (zero,) * _NGRP, unroll=4)
        for q in range(_NGRP):
            out_v[pl.ds(c * _GW + q * 16, 16)] = accs[q]

    issue_idx(0, 0)
    wait_idx(0, 0)
    issue_gathers(0)
    issue_idx(1, 1)

    def pair_body(cp, carry):
        c0 = cp * 2
        drain_gathers(0)

        @pl.when(c0 + 2 < _NCH)
        def _():
            issue_idx(c0 + 2, 0)

        wait_idx(c0 + 1, 1)
        issue_gathers(1)
        compute(c0, 0)
        drain_gathers(1)

        @pl.when(c0 + 3 < _NCH)
        def _():
            issue_idx(c0 + 3, 1)

        @pl.when(c0 + 2 < _NCH)
        def _():
            wait_idx(c0 + 2, 0)
            issue_gathers(0)

        compute(c0 + 1, 1)
        return carry

    lax.fori_loop(0, _NCH // 2, pair_body, 0)

    bv = bv_v[...]
    for g in range(_SPW // 16):
        z = out_v[pl.ds(g * 16, 16)] + bv
        out_v[pl.ds(g * 16, 16)] = 1.0 / (1.0 + jnp.exp(-z))
    pltpu.sync_copy(out_v, out_ref.at[pl.ds(sbase, _SPW)])


_sc_call = functools.partial(
    pl.kernel,
    out_type=jax.ShapeDtypeStruct((_B,), jnp.float32),
    mesh=plsc.VectorSubcoreMesh(core_axis_name="c", subcore_axis_name="s"),
    compiler_params=pltpu.CompilerParams(
        needs_layout_passes=False, use_tc_tiling_on_sc=False),
    scratch_types=[
        pltpu.VMEM((_L, _GW), jnp.int32),
        pltpu.VMEM((_L, _GW), jnp.int32),
        pltpu.VMEM((_L, _GW), jnp.float32),
        pltpu.VMEM((_L, _GW), jnp.float32),
        pltpu.VMEM((_EMB,), jnp.float32),
        pltpu.VMEM((_SPW,), jnp.float32),
        pltpu.SemaphoreType.DMA,
        pltpu.SemaphoreType.DMA,
        pltpu.SemaphoreType.DMA,
        pltpu.SemaphoreType.DMA,
    ],
)(_sc_body)


def kernel(x, table, W, b):
    xt = x.T.astype(jnp.int32)
    tabT = table.T
    wcol = jnp.broadcast_to(
        (W[0].astype(jnp.float32) / jnp.float32(_L))[:, None], (_EMB, 128))
    tw = _tw_call(tabT, wcol)
    bv = jnp.broadcast_to(b.astype(jnp.float32), (_EMB,))
    out = _sc_call(xt, tw, bv)
    return out.reshape(_B, 1)


# speedup vs baseline: 65.8212x; 2.0133x over previous
import functools

import jax
import jax.numpy as jnp
from jax import lax
from jax.experimental import pallas as pl
from jax.experimental.pallas import tpu as pltpu
from jax.experimental.pallas import tpu_sc as plsc

_B = 16384
_L = 200
_EMB = 16
_V = 1_000_000
_NC = 2
_NS = 16
_NW = _NC * _NS
_SPW = _B // _NW
_GW = 128
_NCH = _SPW // _GW
_NGRP = _GW // 16
_H0 = 96
_VP = 1_048_576
_TWC = 131072


def _tw_body(tabT_ref, w_ref, tw_ref):
    w = w_ref[:, 0:1]
    tw_ref[...] = jnp.sum(tabT_ref[...] * w, axis=0)


_tw_call = pl.pallas_call(
    _tw_body,
    grid=(_VP // _TWC,),
    in_specs=[
        pl.BlockSpec((_EMB, _TWC), lambda j: (0, j)),
        pl.BlockSpec((_EMB, 128), lambda j: (0, 0)),
    ],
    out_specs=pl.BlockSpec((_TWC,), lambda j: (j,)),
    out_shape=jax.ShapeDtypeStruct((_VP,), jnp.float32),
)


def _sc_body(xt_ref, tw_ref, b_ref, out_ref,
             idxA, idxB, rowsA, rowsB, bv_v, out_v, stage_v, tw_sh,
             semiA, semiB, semgA, semgB):
    sid = lax.axis_index("s")
    wid = sid * _NC + lax.axis_index("c")
    sbase = wid * _SPW
    pltpu.sync_copy(b_ref, bv_v)

    idxs, rows = (idxA, idxB), (rowsA, rowsB)
    semis, semgs = (semiA, semiB), (semgA, semgB)
    roff, rh = (0, _H0), (_H0, _L - _H0)

    def idx_desc(p, c):
        src = xt_ref.at[pl.ds(roff[p], rh[p]),
                        pl.ds(sbase + c * _GW, _GW)]
        return pltpu.make_async_copy(src, idxs[p], semis[p])

    def issue_idx(p, c):
        idx_desc(p, c).start()

    issue_idx(0, 0)
    issue_idx(1, 0)

    for i in range(8):
        off = (sid + 16 * i) * 8192
        pltpu.sync_copy(tw_ref.at[pl.ds(off, 8192)], stage_v)
        pltpu.sync_copy(stage_v, tw_sh.at[pl.ds(off, 8192)])

    plsc.subcore_barrier()

    def issue_gathers(p):
        def issue(j, carry):
            pltpu.async_copy(
                tw_sh.at[idxs[p].at[j, :]], rows[p].at[j, :], semgs[p])
            return carry

        lax.fori_loop(0, rh[p], issue, 0, unroll=8)

    def drain_gathers(p):
        pltpu.make_async_copy(
            xt_ref.at[pl.ds(roff[p], rh[p]), pl.ds(sbase, _GW)],
            rows[p], semgs[p]).wait()

    def compute(p, c):
        def red(k, accs):
            return tuple(accs[q] + rows[p][k, pl.ds(q * 16, 16)]
                         for q in range(_NGRP))

        zero = jnp.zeros((16,), jnp.float32)
        accs = lax.fori_loop(0, rh[p], red, (zero,) * _NGRP, unroll=4)
        for q in range(_NGRP):
            sl = pl.ds(c * _GW + q * 16, 16)
            if p == 0:
                out_v[sl] = accs[q]
            else:
                out_v[sl] = out_v[sl] + accs[q]

    idx_desc(0, 0).wait()
    issue_gathers(0)
    idx_desc(1, 0).wait()
    issue_gathers(1)

    def chunk_body(c, carry):
        drain_gathers(0)

        @pl.when(c + 1 < _NCH)
        def _():
            issue_idx(0, c + 1)

        compute(0, c)
        drain_gathers(1)

        @pl.when(c + 1 < _NCH)
        def _():
            issue_idx(1, c + 1)
            idx_desc(0, c + 1).wait()
            issue_gathers(0)

        compute(1, c)

        @pl.when(c + 1 < _NCH)
        def _():
            idx_desc(1, c + 1).wait()
            issue_gathers(1)

        return carry

    lax.fori_loop(0, _NCH, chunk_body, 0)

    bv = bv_v[...]
    for g in range(_SPW // 16):
        z = out_v[pl.ds(g * 16, 16)] + bv
        out_v[pl.ds(g * 16, 16)] = 1.0 / (1.0 + jnp.exp(-z))
    pltpu.sync_copy(out_v, out_ref.at[pl.ds(sbase, _SPW)])


_sc_call = functools.partial(
    pl.kernel,
    out_type=jax.ShapeDtypeStruct((_B,), jnp.float32),
    mesh=plsc.VectorSubcoreMesh(core_axis_name="c", subcore_axis_name="s"),
    compiler_params=pltpu.CompilerParams(
        needs_layout_passes=False, use_tc_tiling_on_sc=True),
    scratch_types=[
        pltpu.VMEM((_H0, _GW), jnp.int32),
        pltpu.VMEM((_L - _H0, _GW), jnp.int32),
        pltpu.VMEM((_H0, _GW), jnp.float32),
        pltpu.VMEM((_L - _H0, _GW), jnp.float32),
        pltpu.VMEM((_EMB,), jnp.float32),
        pltpu.VMEM((_SPW,), jnp.float32),
        pltpu.VMEM((8192,), jnp.float32),
        pltpu.VMEM_SHARED((_VP,), jnp.float32),
        pltpu.SemaphoreType.DMA,
        pltpu.SemaphoreType.DMA,
        pltpu.SemaphoreType.DMA,
        pltpu.SemaphoreType.DMA,
    ],
)(_sc_body)


def kernel(x, table, W, b):
    xt = x.T.astype(jnp.int32)
    tabT = table.T
    wcol = jnp.broadcast_to(
        (W[0].astype(jnp.float32) / jnp.float32(_L))[:, None], (_EMB, 128))
    tw = _tw_call(tabT, wcol)
    bv = jnp.broadcast_to(b.astype(jnp.float32), (_EMB,))
    out = _sc_call(xt, tw, bv)
    return out.reshape(_B, 1)
